# baseline (speedup 1.0000x reference)
import jax
import jax.numpy as jnp
import numpy as np
from jax.experimental import pallas as pl

N = 10000
E = 640000
IN = 4
H1 = 512
H2 = 256
OUT = 16
BS = 100
D = 4
NH = 4
FF = 2048
NL = 4


def _gcn_conv(x, src, dst, W, b):
    n = x.shape[0]
    loop = jnp.arange(n, dtype=src.dtype)
    s = jnp.concatenate([src, loop])
    d = jnp.concatenate([dst, loop])
    deg = jnp.zeros((n,), x.dtype).at[d].add(1.0)
    dinv = jax.lax.rsqrt(deg)
    norm = dinv[s] * dinv[d]
    h = x @ W
    out = jnp.zeros_like(h).at[d].add(h[s] * norm[:, None])
    return out + b


def _layer_norm(x, gamma, beta):
    mu = x.mean(axis=-1, keepdims=True)
    var = ((x - mu) ** 2).mean(axis=-1, keepdims=True)
    return (x - mu) / jnp.sqrt(var + 1e-5) * gamma + beta


def _mha(x, in_w, in_b, out_w, out_b):
    S, B, Emb = x.shape
    hd = Emb // NH
    qkv = x @ in_w.T + in_b
    q, k, v = jnp.split(qkv, 3, axis=-1)
    q = q.reshape(S, B, NH, hd)
    k = k.reshape(S, B, NH, hd)
    v = v.reshape(S, B, NH, hd)
    scores = jnp.einsum('sbhd,tbhd->bhst', q, k) / float(np.sqrt(hd))
    attn = jax.nn.softmax(scores, axis=-1)
    o = jnp.einsum('bhst,tbhd->sbhd', attn, v).reshape(S, B, Emb)
    return o @ out_w.T + out_b


def _head_kernel(adj_ref, seq_ref, w_ref, b_ref, out_ref):
    xcat = jnp.concatenate([adj_ref[...], seq_ref[...]], axis=1)
    logits = xcat @ w_ref[...].T + b_ref[...][None, :]
    m = jnp.max(logits, axis=1, keepdims=True)
    e = jnp.exp(logits - m)
    lse = jnp.log(jnp.sum(e, axis=1, keepdims=True))
    out_ref[...] = logits - m - lse


def kernel(seq_x, W1, b1, W2, b2, fc1_w, fc1_b, fc2_w, fc2_b, tin_w, tin_b, tout_w, tout_b, tln1_g, tln1_b, tff1_w, tff1_b, tff2_w, tff2_b, tln2_g, tln2_b, edge_index, batch, batch_size):
    src = edge_index[0]
    dst = edge_index[1]
    h = jax.nn.relu(_gcn_conv(seq_x, src, dst, W1, b1))
    h = jax.nn.relu(_gcn_conv(h, src, dst, W2, b2))
    sums = jax.ops.segment_sum(h, batch, num_segments=BS)
    cnt = jax.ops.segment_sum(jnp.ones((h.shape[0],), h.dtype), batch, num_segments=BS)
    pooled = sums / jnp.maximum(cnt, 1.0)[:, None]
    adj = jax.nn.relu(pooled @ fc1_w.T + fc1_b)
    sx = seq_x.reshape(BS, -1, D)
    for i in range(NL):
        sx = _layer_norm(sx + _mha(sx, tin_w[i], tin_b[i], tout_w[i], tout_b[i]), tln1_g[i], tln1_b[i])
        ff = jax.nn.relu(sx @ tff1_w[i].T + tff1_b[i]) @ tff2_w[i].T + tff2_b[i]
        sx = _layer_norm(sx + ff, tln2_g[i], tln2_b[i])
    seq_feat = sx.mean(axis=1)

    out = pl.pallas_call(
        _head_kernel,
        out_shape=jax.ShapeDtypeStruct((BS, OUT), jnp.float32),
    )(adj, seq_feat, fc2_w, fc2_b)
    return out


# speedup vs baseline: 2.8658x; 2.8658x over previous
import functools

import jax
import jax.numpy as jnp
from jax import lax
from jax.experimental import pallas as pl
from jax.experimental.pallas import tpu as pltpu
from jax.experimental.pallas import tpu_sc as plsc

N = 10000
E = 640000
IN = 4
H1 = 512
H2 = 256
OUT = 16
BS = 100
D = 4
NH = 4
FF = 2048
NL = 4

EB1 = 2000
EB2 = 2560

_mesh = plsc.VectorSubcoreMesh(core_axis_name="c", subcore_axis_name="s")


def _matT(a, w):
    return lax.dot_general(a, w, (((1,), (1,)), ((), ())),
                           preferred_element_type=jnp.float32)


@functools.partial(
    pl.kernel,
    out_type=jax.ShapeDtypeStruct((32, N), jnp.float32),
    mesh=_mesh,
    compiler_params=pltpu.CompilerParams(needs_layout_passes=False),
    scratch_types=[
        pltpu.VMEM((EB1,), jnp.int32),
        pltpu.VMEM((N,), jnp.float32),
    ],
)
def _deg_sc(edst, zeros_n, out, dst_v, hist):
    cid = lax.axis_index("c")
    sid = lax.axis_index("s")
    wid = cid * 16 + sid
    pltpu.sync_copy(zeros_n, hist)
    ones16 = jnp.ones((16,), jnp.float32)

    def body(i, _):
        base = wid * (E // 32) + i * EB1
        pltpu.sync_copy(edst.at[pl.ds(base, EB1)], dst_v)

        def inner(u, _):
            d16 = dst_v[pl.ds(u * 16, 16)]
            plsc.addupdate_scatter(hist, [d16], ones16)
            return 0

        lax.fori_loop(0, EB1 // 16, inner, 0)
        return 0

    lax.fori_loop(0, (E // 32) // EB1, body, 0)
    pltpu.sync_copy(hist, out.at[wid])


def _prep_tc(deg_ref, x_ref, dinv_ref, xs_ref):
    deg = jnp.sum(deg_ref[...], axis=0) + 1.0
    dinv = lax.rsqrt(deg)[:, None]
    dinv_ref[...] = dinv
    xs_ref[...] = dinv * x_ref[...]


@functools.partial(
    pl.kernel,
    out_type=jax.ShapeDtypeStruct((32, N * IN), jnp.float32),
    mesh=_mesh,
    compiler_params=pltpu.CompilerParams(needs_layout_passes=False),
    scratch_types=[
        pltpu.VMEM((EB1,), jnp.int32),
        pltpu.VMEM((EB1,), jnp.int32),
        pltpu.VMEM((N * IN,), jnp.float32),
        pltpu.VMEM((N * IN,), jnp.float32),
    ],
)
def _conv1_sc(xs, esrc, edst, zeros_n4, out, src_v, dst_v, xs_v, acc):
    cid = lax.axis_index("c")
    sid = lax.axis_index("s")
    wid = cid * 16 + sid
    pltpu.sync_copy(zeros_n4, acc)
    pltpu.sync_copy(xs, xs_v)

    def body(i, _):
        base = wid * (E // 32) + i * EB1
        pltpu.sync_copy(esrc.at[pl.ds(base, EB1)], src_v)
        pltpu.sync_copy(edst.at[pl.ds(base, EB1)], dst_v)

        def inner(u, _):
            s4 = src_v[pl.ds(u * 16, 16)] * IN
            d4 = dst_v[pl.ds(u * 16, 16)] * IN
            for f in range(IN):
                v = plsc.load_gather(xs_v, [s4 + f])
                plsc.addupdate_scatter(acc, [d4 + f], v)
            return 0

        lax.fori_loop(0, EB1 // 16, inner, 0)
        return 0

    lax.fori_loop(0, (E // 32) // EB1, body, 0)
    pltpu.sync_copy(acc, out.at[wid])


def _dense_tc(p1_ref, xs_ref, dinv_ref, w1_ref, b1_ref, w2_ref, out_ref):
    p1 = jnp.sum(p1_ref[...], axis=0)
    xs = xs_ref[...]
    dinv = dinv_ref[...]
    t = dinv * (p1 + xs)
    h1 = jnp.maximum(jnp.dot(t, w1_ref[...],
                             preferred_element_type=jnp.float32)
                     + b1_ref[...], 0.0)
    y = jnp.dot(h1, w2_ref[...], preferred_element_type=jnp.float32)
    out_ref[...] = dinv * y


NSL = H1 // IN


@functools.partial(
    pl.kernel,
    out_type=jax.ShapeDtypeStruct((NSL, N * IN), jnp.float32),
    mesh=_mesh,
    compiler_params=pltpu.CompilerParams(needs_layout_passes=False),
    scratch_types=[
        pltpu.VMEM((EB2,), jnp.int32),
        pltpu.VMEM((EB2,), jnp.int32),
        pltpu.VMEM((N * IN,), jnp.float32),
        pltpu.VMEM((N * IN,), jnp.float32),
    ],
)
def _conv2_sc(ysfine, esrc, edst, zeros_n4, out, src_v, dst_v, tab_v, acc):
    cid = lax.axis_index("c")
    sid = lax.axis_index("s")
    wid = cid * 16 + sid

    for r in range(NSL // 32):
        k = r * 32 + wid
        pltpu.sync_copy(ysfine.at[k], tab_v)
        pltpu.sync_copy(zeros_n4, acc)

        def body(i, _):
            base = i * EB2
            pltpu.sync_copy(esrc.at[pl.ds(base, EB2)], src_v)
            pltpu.sync_copy(edst.at[pl.ds(base, EB2)], dst_v)

            def inner(u, _):
                s4 = src_v[pl.ds(u * 16, 16)] * IN
                d4 = dst_v[pl.ds(u * 16, 16)] * IN
                for f in range(IN):
                    v = plsc.load_gather(tab_v, [s4 + f])
                    plsc.addupdate_scatter(acc, [d4 + f], v)
                return 0

            lax.fori_loop(0, EB2 // 16, inner, 0)
            return 0

        lax.fori_loop(0, E // EB2, body, 0)
        pltpu.sync_copy(acc, out.at[k])


def _pool_tc(z_ref, ys_ref, dinv_ref, b2_ref, batch_ref, fc1w_ref, fc1b_ref,
             out_ref, sums_acc, cnt_acc):
    n = pl.program_id(0)
    h2 = jnp.maximum(dinv_ref[...] * (z_ref[...] + ys_ref[...])
                     + b2_ref[...], 0.0)
    bvec = batch_ref[0, 0, :]
    iota = lax.broadcasted_iota(jnp.int32, (BS, h2.shape[0]), 0)
    onehot = (iota == bvec[None, :]).astype(jnp.float32)

    @pl.when(n == 0)
    def _():
        sums_acc[...] = jnp.zeros_like(sums_acc)
        cnt_acc[...] = jnp.zeros_like(cnt_acc)

    sums_acc[...] += jnp.dot(onehot, h2, preferred_element_type=jnp.float32)
    cnt_acc[...] += jnp.dot(onehot, jnp.ones((h2.shape[0], 128), jnp.float32),
                            preferred_element_type=jnp.float32)

    @pl.when(n == pl.num_programs(0) - 1)
    def _():
        cnt = jnp.maximum(cnt_acc[:, 0:1], 1.0)
        pooled = sums_acc[...] / cnt
        out_ref[...] = jnp.maximum(_matT(pooled, fc1w_ref[...])
                                   + fc1b_ref[...], 0.0)


def _tr_tc(x_ref, adj_ref, tin_w, tin_b, tout_w, tout_b, tln1_g, tln1_b,
           tff1_w, tff1_b, tff2_w, tff2_b, tln2_g, tln2_b, fc2w_ref,
           fc2b_ref, out_ref, sf_acc):
    bstep = pl.program_id(0)
    x = x_ref[...]
    TB = x.shape[0]
    R = TB * BS

    def lnorm(z, g, b):
        mu = jnp.mean(z, axis=1, keepdims=True)
        var = jnp.mean((z - mu) ** 2, axis=1, keepdims=True)
        return (z - mu) * lax.rsqrt(var + 1e-5) * g[None, :] + b[None, :]

    for i in range(NL):
        flat = x.reshape(R, D)
        qkv = (_matT(flat, tin_w[i]) + tin_b[i][None, :]).reshape(TB, BS,
                                                                  3 * D)
        outs = []
        for h in range(NH):
            qh = qkv[:, :, h]
            kh = qkv[:, :, NH + h]
            vh = qkv[:, :, 2 * NH + h]
            scores = qh[:, :, None] * kh[:, None, :]
            m = jnp.max(scores, axis=2, keepdims=True)
            e = jnp.exp(scores - m)
            a = e / jnp.sum(e, axis=2, keepdims=True)
            outs.append(jnp.sum(a * vh[:, None, :], axis=2)[:, :, None])
        o = jnp.concatenate(outs, axis=2)
        attn_out = _matT(o.reshape(R, D), tout_w[i]) + tout_b[i][None, :]
        z1 = lnorm(flat + attn_out, tln1_g[i], tln1_b[i])
        ff = _matT(jnp.maximum(_matT(z1, tff1_w[i]) + tff1_b[i][None, :],
                               0.0), tff2_w[i]) + tff2_b[i][None, :]
        z2 = lnorm(z1 + ff, tln2_g[i], tln2_b[i])
        x = z2.reshape(TB, BS, D)

    partial = jnp.sum(x, axis=0) * (1.0 / BS)

    @pl.when(bstep == 0)
    def _():
        sf_acc[...] = jnp.zeros_like(sf_acc)

    sf_acc[...] += partial

    @pl.when(bstep == pl.num_programs(0) - 1)
    def _():
        xcat = jnp.concatenate([adj_ref[...], sf_acc[...]], axis=1)
        logits = _matT(xcat, fc2w_ref[...]) + fc2b_ref[...]
        mx = jnp.max(logits, axis=1, keepdims=True)
        ex = jnp.exp(logits - mx)
        lse = jnp.log(jnp.sum(ex, axis=1, keepdims=True))
        out_ref[...] = logits - mx - lse


def kernel(seq_x, W1, b1, W2, b2, fc1_w, fc1_b, fc2_w, fc2_b, tin_w, tin_b,
           tout_w, tout_b, tln1_g, tln1_b, tff1_w, tff1_b, tff2_w, tff2_b,
           tln2_g, tln2_b, edge_index, batch, batch_size):
    f32 = jnp.float32
    esrc = edge_index[0].astype(jnp.int32)
    edst = edge_index[1].astype(jnp.int32)

    deg32 = _deg_sc(edst, jnp.zeros((N,), f32))

    dinv, xs = pl.pallas_call(
        _prep_tc,
        out_shape=(jax.ShapeDtypeStruct((N, 1), f32),
                   jax.ShapeDtypeStruct((N, IN), f32)),
    )(deg32, seq_x)

    xs_flat = xs.reshape(N * IN)
    p1raw = _conv1_sc(xs_flat, esrc, edst, jnp.zeros((N * IN,), f32))

    ys = pl.pallas_call(
        _dense_tc,
        grid=(10,),
        in_specs=[
            pl.BlockSpec((32, N // 10, IN), lambda n: (0, n, 0)),
            pl.BlockSpec((N // 10, IN), lambda n: (n, 0)),
            pl.BlockSpec((N // 10, 1), lambda n: (n, 0)),
            pl.BlockSpec((IN, H1), lambda n: (0, 0)),
            pl.BlockSpec((1, H1), lambda n: (0, 0)),
            pl.BlockSpec((H1, H1), lambda n: (0, 0)),
        ],
        out_specs=pl.BlockSpec((N // 10, H1), lambda n: (n, 0)),
        out_shape=jax.ShapeDtypeStruct((N, H1), f32),
    )(p1raw.reshape(32, N, IN), xs, dinv, W1, b1.reshape(1, H1), W2)

    ys_fine = jnp.transpose(ys.reshape(N, NSL, IN), (1, 0, 2)).reshape(NSL,
                                                                       N * IN)
    zraw = _conv2_sc(ys_fine, esrc, edst, jnp.zeros((N * IN,), f32))
    ztr = jnp.transpose(zraw.reshape(NSL, N, IN), (1, 0, 2)).reshape(N, H1)

    adj = pl.pallas_call(
        _pool_tc,
        grid=(10,),
        in_specs=[
            pl.BlockSpec((N // 10, H1), lambda n: (n, 0)),
            pl.BlockSpec((N // 10, H1), lambda n: (n, 0)),
            pl.BlockSpec((N // 10, 1), lambda n: (n, 0)),
            pl.BlockSpec((1, H1), lambda n: (0, 0)),
            pl.BlockSpec((1, 1, N // 10), lambda n: (n, 0, 0)),
            pl.BlockSpec((H2, H1), lambda n: (0, 0)),
            pl.BlockSpec((1, H2), lambda n: (0, 0)),
        ],
        out_specs=pl.BlockSpec((BS, H2), lambda n: (0, 0)),
        out_shape=jax.ShapeDtypeStruct((BS, H2), f32),
        scratch_shapes=[pltpu.VMEM((BS, H1), f32), pltpu.VMEM((BS, 128), f32)],
    )(ztr, ys, dinv, b2.reshape(1, H1), batch.reshape(10, 1, N // 10),
      fc1_w, fc1_b.reshape(1, H2))

    xT = jnp.transpose(seq_x.reshape(BS, BS, D), (1, 0, 2))
    TB = 4
    wspec = lambda shp: pl.BlockSpec(shp, lambda b: (0,) * len(shp))
    out = pl.pallas_call(
        _tr_tc,
        grid=(BS // TB,),
        in_specs=[
            pl.BlockSpec((TB, BS, D), lambda b: (b, 0, 0)),
            wspec((BS, H2)),
            wspec((NL, 3 * D, D)), wspec((NL, 3 * D)),
            wspec((NL, D, D)), wspec((NL, D)),
            wspec((NL, D)), wspec((NL, D)),
            wspec((NL, FF, D)), wspec((NL, FF)),
            wspec((NL, D, FF)), wspec((NL, D)),
            wspec((NL, D)), wspec((NL, D)),
            wspec((OUT, H2 + D)), wspec((1, OUT)),
        ],
        out_specs=pl.BlockSpec((BS, OUT), lambda b: (0, 0)),
        out_shape=jax.ShapeDtypeStruct((BS, OUT), f32),
        scratch_shapes=[pltpu.VMEM((BS, D), f32)],
    )(xT, adj, tin_w, tin_b, tout_w, tout_b, tln1_g, tln1_b,
      tff1_w, tff1_b, tff2_w, tff2_b, tln2_g, tln2_b,
      fc2_w, fc2_b.reshape(1, OUT))
    return out
